# baseline (speedup 1.0000x reference)
import functools
import math

import jax
import jax.numpy as jnp
from jax import lax
from jax.experimental import pallas as pl
from jax.experimental.pallas import tpu as pltpu
from jax.experimental.pallas import tpu_sc as plsc

N = 100000
E = 1600000
D = 128
H = 128
BN_SCALE = 1.0 / math.sqrt(1.0 + 1e-3)

R = 10240
NRANGE = 10
TPS = R // 16
C = 512
DUMMY = R

BLK = 2000


def _prep_body(x_ref, s1_ref, t1_ref, w1_ref, b1_ref, s2_ref, t2_ref,
               w2_ref, b2_ref, o_ref):
    h = x_ref[...] * s1_ref[...] + t1_ref[...]
    h = jax.nn.gelu(jnp.dot(h, w1_ref[...], preferred_element_type=jnp.float32)
                    + b1_ref[...])
    h = h * s2_ref[...] + t2_ref[...]
    o_ref[...] = jax.nn.gelu(
        jnp.dot(h, w2_ref[...], preferred_element_type=jnp.float32) + b2_ref[...])


def _row_spec(rows, cols):
    return pl.BlockSpec((rows, cols), lambda i: (i, 0))


def _full_spec(shape):
    return pl.BlockSpec(shape, lambda i: (0,) * len(shape))


def _prep_ffn(x, s1, t1, w1, b1, s2, t2, w2, b2):
    grid = (N // BLK,)
    return pl.pallas_call(
        _prep_body,
        grid=grid,
        in_specs=[
            _row_spec(BLK, D),
            _full_spec((1, D)), _full_spec((1, D)),
            _full_spec((D, H)), _full_spec((1, H)),
            _full_spec((1, H)), _full_spec((1, H)),
            _full_spec((H, H)), _full_spec((1, H)),
        ],
        out_specs=_row_spec(BLK, H),
        out_shape=jax.ShapeDtypeStruct((N, H), jnp.float32),
    )(x, s1, t1, w1, b1, s2, t2, w2, b2)


def _upd_body(x_ref, a_ref, s1x_ref, t1x_ref, s1a_ref, t1a_ref,
              w1x_ref, w1a_ref, b1_ref, s2_ref, t2_ref, w2_ref, b2_ref, o_ref):
    xs = x_ref[...] * s1x_ref[...] + t1x_ref[...]
    aa = a_ref[...] * s1a_ref[...] + t1a_ref[...]
    h = (jnp.dot(xs, w1x_ref[...], preferred_element_type=jnp.float32)
         + jnp.dot(aa, w1a_ref[...], preferred_element_type=jnp.float32)
         + b1_ref[...])
    h = jax.nn.gelu(h)
    h = h * s2_ref[...] + t2_ref[...]
    h = jax.nn.gelu(jnp.dot(h, w2_ref[...], preferred_element_type=jnp.float32)
                    + b2_ref[...])
    norm = jnp.sqrt(jnp.sum(h * h, axis=-1, keepdims=True))
    o_ref[...] = h / jnp.maximum(norm, 1e-12)


def _upd_ffn(x, agg, s1x, t1x, s1a, t1a, w1x, w1a, b1, s2, t2, w2, b2):
    grid = (N // BLK,)
    return pl.pallas_call(
        _upd_body,
        grid=grid,
        in_specs=[
            _row_spec(BLK, D),
            _row_spec(BLK, H),
            _full_spec((1, D)), _full_spec((1, D)),
            _full_spec((1, H)), _full_spec((1, H)),
            _full_spec((D, H)), _full_spec((H, H)), _full_spec((1, H)),
            _full_spec((1, H)), _full_spec((1, H)),
            _full_spec((H, H)), _full_spec((1, H)),
        ],
        out_specs=_row_spec(BLK, H),
        out_shape=jax.ShapeDtypeStruct((N, H), jnp.float32),
    )(x, agg, s1x, t1x, s1a, t1a, w1x, w1a, b1, s2, t2, w2, b2)


def _edge_agg(tab, dst_arr, src_arr, ew):
    mesh = plsc.VectorSubcoreMesh(core_axis_name="c", subcore_axis_name="s")

    @functools.partial(
        pl.kernel,
        out_type=jax.ShapeDtypeStruct((NRANGE * R, H), jnp.float32),
        mesh=mesh,
        scratch_types=[
            pltpu.VMEM((C,), jnp.int32),
            pltpu.VMEM((C,), jnp.int32),
            pltpu.VMEM((C,), jnp.float32),
            pltpu.VMEM((128,), jnp.int32),
            pltpu.VMEM((128,), jnp.int32),
            pltpu.VMEM((128,), jnp.int32),
            pltpu.VMEM((128,), jnp.int32),
            pltpu.VMEM((C,), jnp.int32),
            pltpu.VMEM((128, H), jnp.float32),
            pltpu.VMEM((160, H), jnp.float32),
            pltpu.VMEM_SHARED((R + 16, H), jnp.float32),
            pltpu.SemaphoreType.DMA,
        ],
    )
    def k(tab_hbm, dst_hbm, src_hbm, ew_hbm, out_hbm,
          dst_c, src_c, w_c, gi0, gi1, gi2, gi3, dloc, rows, zeros, acc, sem):
        cid = lax.axis_index("c")
        sid = lax.axis_index("s")
        zero16f = jnp.zeros((16,), jnp.float32)

        def zinit(j, _):
            def zf(f, __):
                zeros[j, pl.ds(f * 16, 16)] = zero16f
                return 0
            return lax.fori_loop(0, H // 16, zf, 0)
        lax.fori_loop(0, 160, zinit, 0)

        nch = 195 + (jnp.right_shift(sid - 5, 31) & 1)

        for rr in range(NRANGE // 2):
            rid = rr * 2 + cid
            lo = rid * R
            for kk in range(TPS // 160):
                pltpu.sync_copy(zeros, acc.at[pl.ds(sid * TPS + kk * 160, 160)])
            plsc.subcore_barrier()

            def chunk_body(ch, _):
                base = (ch * 16 + sid) * C
                pltpu.sync_copy(dst_hbm.at[pl.ds(base, C)], dst_c)
                pltpu.sync_copy(src_hbm.at[pl.ds(base, C)], src_c)
                pltpu.sync_copy(ew_hbm.at[pl.ds(base, C)], w_c)

                for t, gi in enumerate((gi0, gi1, gi2, gi3)):
                    def vf(i, __, t=t, gi=gi):
                        j = t * 8 + i
                        d = dst_c[pl.ds(j * 16, 16)]
                        s = src_c[pl.ds(j * 16, 16)]
                        dl = d - lo
                        hit = (jnp.right_shift(dl - R, 31)
                               & ~jnp.right_shift(dl, 31) & 1)
                        gi[pl.ds(i * 16, 16)] = hit * s
                        dloc[pl.ds(j * 16, 16)] = hit * dl + (1 - hit) * DUMMY
                        return 0
                    lax.fori_loop(0, 8, vf, 0)

                for t, gi in enumerate((gi0, gi1, gi2, gi3)):
                    tb = t * 128
                    pltpu.async_copy(tab_hbm.at[gi], rows, sem).wait()

                    def group(g, ___, tb=tb):
                        gb = tb + g * 16
                        dl16 = dloc[pl.ds(gb, 16)]
                        w16 = w_c[pl.ds(gb, 16)]
                        for l in range(16):
                            ws = w16[l]
                            r0 = g * 16 + l
                            for f in range(H // 16):
                                rows[r0, pl.ds(f * 16, 16)] = (
                                    rows[r0, pl.ds(f * 16, 16)] * ws)
                        pltpu.sync_copy(rows.at[pl.ds(g * 16, 16)],
                                        acc.at[dl16], add=True)
                        return 0
                    lax.fori_loop(0, 8, group, 0)
                return 0
            lax.fori_loop(0, nch, chunk_body, 0)
            plsc.subcore_barrier()

            for kk in range(TPS // 160):
                off = sid * TPS + kk * 160
                pltpu.sync_copy(acc.at[pl.ds(off, 160)],
                                out_hbm.at[pl.ds(lo + off, 160)])
            plsc.subcore_barrier()

    return k(tab, dst_arr, src_arr, ew)


def kernel(node_representations, edges, edge_weights,
           prep_bn1_gamma, prep_bn1_beta, prep_dense1_W, prep_dense1_b,
           prep_bn2_gamma, prep_bn2_beta, prep_dense2_W, prep_dense2_b,
           upd_bn1_gamma, upd_bn1_beta, upd_dense1_W, upd_dense1_b,
           upd_bn2_gamma, upd_bn2_beta, upd_dense2_W, upd_dense2_b):
    f32 = jnp.float32

    prep = _prep_ffn(
        node_representations,
        (prep_bn1_gamma * BN_SCALE)[None, :].astype(f32),
        prep_bn1_beta[None, :],
        prep_dense1_W, prep_dense1_b[None, :],
        (prep_bn2_gamma * BN_SCALE)[None, :].astype(f32),
        prep_bn2_beta[None, :],
        prep_dense2_W, prep_dense2_b[None, :],
    )

    agg = _edge_agg(prep, edges[0], edges[1], edge_weights)

    out = _upd_ffn(
        node_representations, agg,
        (upd_bn1_gamma[:D] * BN_SCALE)[None, :].astype(f32),
        upd_bn1_beta[None, :D],
        (upd_bn1_gamma[D:] * BN_SCALE)[None, :].astype(f32),
        upd_bn1_beta[None, D:],
        upd_dense1_W[:D], upd_dense1_W[D:],
        upd_dense1_b[None, :],
        (upd_bn2_gamma * BN_SCALE)[None, :].astype(f32),
        upd_bn2_beta[None, :],
        upd_dense2_W, upd_dense2_b[None, :],
    )
    return out


# speedup vs baseline: 1.0001x; 1.0001x over previous
import functools
import math

import jax
import jax.numpy as jnp
from jax import lax
from jax.experimental import pallas as pl
from jax.experimental.pallas import tpu as pltpu
from jax.experimental.pallas import tpu_sc as plsc

N = 100000
E = 1600000
D = 128
H = 128
BN_SCALE = 1.0 / math.sqrt(1.0 + 1e-3)

R = 10240
NRANGE = 10
TPS = R // 16
C = 512
DUMMY = R

BLK = 2000


def _prep_body(x_ref, s1_ref, t1_ref, w1_ref, b1_ref, s2_ref, t2_ref,
               w2_ref, b2_ref, o_ref):
    h = x_ref[...] * s1_ref[...] + t1_ref[...]
    h = jax.nn.gelu(jnp.dot(h, w1_ref[...], preferred_element_type=jnp.float32)
                    + b1_ref[...])
    h = h * s2_ref[...] + t2_ref[...]
    o_ref[...] = jax.nn.gelu(
        jnp.dot(h, w2_ref[...], preferred_element_type=jnp.float32) + b2_ref[...])


def _row_spec(rows, cols):
    return pl.BlockSpec((rows, cols), lambda i: (i, 0))


def _full_spec(shape):
    return pl.BlockSpec(shape, lambda i: (0,) * len(shape))


def _prep_ffn(x, s1, t1, w1, b1, s2, t2, w2, b2):
    grid = (N // BLK,)
    return pl.pallas_call(
        _prep_body,
        grid=grid,
        in_specs=[
            _row_spec(BLK, D),
            _full_spec((1, D)), _full_spec((1, D)),
            _full_spec((D, H)), _full_spec((1, H)),
            _full_spec((1, H)), _full_spec((1, H)),
            _full_spec((H, H)), _full_spec((1, H)),
        ],
        out_specs=_row_spec(BLK, H),
        out_shape=jax.ShapeDtypeStruct((N, H), jnp.float32),
    )(x, s1, t1, w1, b1, s2, t2, w2, b2)


def _upd_body(x_ref, a_ref, s1x_ref, t1x_ref, s1a_ref, t1a_ref,
              w1x_ref, w1a_ref, b1_ref, s2_ref, t2_ref, w2_ref, b2_ref, o_ref):
    xs = x_ref[...] * s1x_ref[...] + t1x_ref[...]
    aa = a_ref[...] * s1a_ref[...] + t1a_ref[...]
    h = (jnp.dot(xs, w1x_ref[...], preferred_element_type=jnp.float32)
         + jnp.dot(aa, w1a_ref[...], preferred_element_type=jnp.float32)
         + b1_ref[...])
    h = jax.nn.gelu(h)
    h = h * s2_ref[...] + t2_ref[...]
    h = jax.nn.gelu(jnp.dot(h, w2_ref[...], preferred_element_type=jnp.float32)
                    + b2_ref[...])
    norm = jnp.sqrt(jnp.sum(h * h, axis=-1, keepdims=True))
    o_ref[...] = h / jnp.maximum(norm, 1e-12)


def _upd_ffn(x, agg, s1x, t1x, s1a, t1a, w1x, w1a, b1, s2, t2, w2, b2):
    grid = (N // BLK,)
    return pl.pallas_call(
        _upd_body,
        grid=grid,
        in_specs=[
            _row_spec(BLK, D),
            _row_spec(BLK, H),
            _full_spec((1, D)), _full_spec((1, D)),
            _full_spec((1, H)), _full_spec((1, H)),
            _full_spec((D, H)), _full_spec((H, H)), _full_spec((1, H)),
            _full_spec((1, H)), _full_spec((1, H)),
            _full_spec((H, H)), _full_spec((1, H)),
        ],
        out_specs=_row_spec(BLK, H),
        out_shape=jax.ShapeDtypeStruct((N, H), jnp.float32),
    )(x, agg, s1x, t1x, s1a, t1a, w1x, w1a, b1, s2, t2, w2, b2)


def _edge_agg(tab, dst_arr, src_arr, ew):
    mesh = plsc.VectorSubcoreMesh(core_axis_name="c", subcore_axis_name="s")

    @functools.partial(
        pl.kernel,
        out_type=jax.ShapeDtypeStruct((NRANGE * R, H), jnp.float32),
        mesh=mesh,
        compiler_params=pltpu.CompilerParams(use_tc_tiling_on_sc=False),
        scratch_types=[
            pltpu.VMEM((C,), jnp.int32),
            pltpu.VMEM((C,), jnp.int32),
            pltpu.VMEM((C,), jnp.float32),
            pltpu.VMEM((C,), jnp.int32),
            pltpu.VMEM((C,), jnp.int32),
            pltpu.VMEM((128, H), jnp.float32),
            pltpu.VMEM((160, H), jnp.float32),
            pltpu.VMEM_SHARED((R + 16, H), jnp.float32),
            pltpu.SemaphoreType.DMA,
        ],
    )
    def k(tab_hbm, dst_hbm, src_hbm, ew_hbm, out_hbm,
          dst_c, src_c, w_c, gidx, dloc, rows, zeros, acc, sem):
        cid = lax.axis_index("c")
        sid = lax.axis_index("s")
        zero16f = jnp.zeros((16,), jnp.float32)

        def zinit(j, _):
            def zf(f, __):
                zeros[j, pl.ds(f * 16, 16)] = zero16f
                return 0
            return lax.fori_loop(0, H // 16, zf, 0)
        lax.fori_loop(0, 160, zinit, 0)

        nch = 195 + (jnp.right_shift(sid - 5, 31) & 1)

        for rr in range(NRANGE // 2):
            rid = rr * 2 + cid
            lo = rid * R
            for kk in range(TPS // 160):
                pltpu.sync_copy(zeros, acc.at[pl.ds(sid * TPS + kk * 160, 160)])
            plsc.subcore_barrier()

            def chunk_body(ch, _):
                base = (ch * 16 + sid) * C
                pltpu.sync_copy(dst_hbm.at[pl.ds(base, C)], dst_c)
                pltpu.sync_copy(src_hbm.at[pl.ds(base, C)], src_c)
                pltpu.sync_copy(ew_hbm.at[pl.ds(base, C)], w_c)

                def vf(i, __):
                    d = dst_c[pl.ds(i * 16, 16)]
                    s = src_c[pl.ds(i * 16, 16)]
                    dl = d - lo
                    hit = (jnp.right_shift(dl - R, 31)
                           & ~jnp.right_shift(dl, 31) & 1)
                    gidx[pl.ds(i * 16, 16)] = hit * s
                    dloc[pl.ds(i * 16, 16)] = hit * dl + (1 - hit) * DUMMY
                    return 0
                lax.fori_loop(0, C // 16, vf, 0)

                def sub(t, __):
                    tb = t * 128

                    def fire(p, ___):
                        gv = gidx[pl.ds(tb + p * 16, 16)]
                        for l in range(16):
                            pltpu.async_copy(tab_hbm.at[gv[l]],
                                             rows.at[p * 16 + l], sem)
                        return 0
                    lax.fori_loop(0, 8, fire, 0)

                    def dr(p, ___):
                        pltpu.make_async_copy(tab_hbm.at[0],
                                              rows.at[0], sem).wait()
                        return 0
                    lax.fori_loop(0, 128, dr, 0)

                    def group(g, ___):
                        gb = tb + g * 16
                        dl16 = dloc[pl.ds(gb, 16)]
                        w16 = w_c[pl.ds(gb, 16)]
                        for l in range(16):
                            ws = w16[l]
                            r0 = g * 16 + l
                            for f in range(H // 16):
                                rows[r0, pl.ds(f * 16, 16)] = (
                                    rows[r0, pl.ds(f * 16, 16)] * ws)
                        pltpu.sync_copy(rows.at[pl.ds(g * 16, 16)],
                                        acc.at[dl16], add=True)
                        return 0
                    lax.fori_loop(0, 8, group, 0)
                    return 0
                lax.fori_loop(0, C // 128, sub, 0)
                return 0
            lax.fori_loop(0, nch, chunk_body, 0)
            plsc.subcore_barrier()

            for kk in range(TPS // 160):
                off = sid * TPS + kk * 160
                pltpu.sync_copy(acc.at[pl.ds(off, 160)],
                                out_hbm.at[pl.ds(lo + off, 160)])
            plsc.subcore_barrier()

    return k(tab, dst_arr, src_arr, ew)


def kernel(node_representations, edges, edge_weights,
           prep_bn1_gamma, prep_bn1_beta, prep_dense1_W, prep_dense1_b,
           prep_bn2_gamma, prep_bn2_beta, prep_dense2_W, prep_dense2_b,
           upd_bn1_gamma, upd_bn1_beta, upd_dense1_W, upd_dense1_b,
           upd_bn2_gamma, upd_bn2_beta, upd_dense2_W, upd_dense2_b):
    f32 = jnp.float32

    prep = _prep_ffn(
        node_representations,
        (prep_bn1_gamma * BN_SCALE)[None, :].astype(f32),
        prep_bn1_beta[None, :],
        prep_dense1_W, prep_dense1_b[None, :],
        (prep_bn2_gamma * BN_SCALE)[None, :].astype(f32),
        prep_bn2_beta[None, :],
        prep_dense2_W, prep_dense2_b[None, :],
    )

    agg = _edge_agg(prep, edges[0], edges[1], edge_weights)

    out = _upd_ffn(
        node_representations, agg,
        (upd_bn1_gamma[:D] * BN_SCALE)[None, :].astype(f32),
        upd_bn1_beta[None, :D],
        (upd_bn1_gamma[D:] * BN_SCALE)[None, :].astype(f32),
        upd_bn1_beta[None, D:],
        upd_dense1_W[:D], upd_dense1_W[D:],
        upd_dense1_b[None, :],
        (upd_bn2_gamma * BN_SCALE)[None, :].astype(f32),
        upd_bn2_beta[None, :],
        upd_dense2_W, upd_dense2_b[None, :],
    )
    return out


# speedup vs baseline: 1.0004x; 1.0003x over previous
import functools
import math

import jax
import jax.numpy as jnp
from jax import lax
from jax.experimental import pallas as pl
from jax.experimental.pallas import tpu as pltpu
from jax.experimental.pallas import tpu_sc as plsc

N = 100000
E = 1600000
D = 128
H = 128
BN_SCALE = 1.0 / math.sqrt(1.0 + 1e-3)

R = 10240
NRANGE = 10
TPS = R // 16
C = 512
DUMMY = R

BLK = 2000


def _prep_body(x_ref, s1_ref, t1_ref, w1_ref, b1_ref, s2_ref, t2_ref,
               w2_ref, b2_ref, o_ref):
    h = x_ref[...] * s1_ref[...] + t1_ref[...]
    h = jax.nn.gelu(jnp.dot(h, w1_ref[...], preferred_element_type=jnp.float32)
                    + b1_ref[...])
    h = h * s2_ref[...] + t2_ref[...]
    o_ref[...] = jax.nn.gelu(
        jnp.dot(h, w2_ref[...], preferred_element_type=jnp.float32) + b2_ref[...])


def _row_spec(rows, cols):
    return pl.BlockSpec((rows, cols), lambda i: (i, 0))


def _full_spec(shape):
    return pl.BlockSpec(shape, lambda i: (0,) * len(shape))


def _prep_ffn(x, s1, t1, w1, b1, s2, t2, w2, b2):
    grid = (N // BLK,)
    return pl.pallas_call(
        _prep_body,
        grid=grid,
        in_specs=[
            _row_spec(BLK, D),
            _full_spec((1, D)), _full_spec((1, D)),
            _full_spec((D, H)), _full_spec((1, H)),
            _full_spec((1, H)), _full_spec((1, H)),
            _full_spec((H, H)), _full_spec((1, H)),
        ],
        out_specs=_row_spec(BLK, H),
        out_shape=jax.ShapeDtypeStruct((N, H), jnp.float32),
    )(x, s1, t1, w1, b1, s2, t2, w2, b2)


def _upd_body(x_ref, a_ref, s1x_ref, t1x_ref, s1a_ref, t1a_ref,
              w1x_ref, w1a_ref, b1_ref, s2_ref, t2_ref, w2_ref, b2_ref, o_ref):
    xs = x_ref[...] * s1x_ref[...] + t1x_ref[...]
    aa = a_ref[...] * s1a_ref[...] + t1a_ref[...]
    h = (jnp.dot(xs, w1x_ref[...], preferred_element_type=jnp.float32)
         + jnp.dot(aa, w1a_ref[...], preferred_element_type=jnp.float32)
         + b1_ref[...])
    h = jax.nn.gelu(h)
    h = h * s2_ref[...] + t2_ref[...]
    h = jax.nn.gelu(jnp.dot(h, w2_ref[...], preferred_element_type=jnp.float32)
                    + b2_ref[...])
    norm = jnp.sqrt(jnp.sum(h * h, axis=-1, keepdims=True))
    o_ref[...] = h / jnp.maximum(norm, 1e-12)


def _upd_ffn(x, agg, s1x, t1x, s1a, t1a, w1x, w1a, b1, s2, t2, w2, b2):
    grid = (N // BLK,)
    return pl.pallas_call(
        _upd_body,
        grid=grid,
        in_specs=[
            _row_spec(BLK, D),
            _row_spec(BLK, H),
            _full_spec((1, D)), _full_spec((1, D)),
            _full_spec((1, H)), _full_spec((1, H)),
            _full_spec((D, H)), _full_spec((H, H)), _full_spec((1, H)),
            _full_spec((1, H)), _full_spec((1, H)),
            _full_spec((H, H)), _full_spec((1, H)),
        ],
        out_specs=_row_spec(BLK, H),
        out_shape=jax.ShapeDtypeStruct((N, H), jnp.float32),
    )(x, agg, s1x, t1x, s1a, t1a, w1x, w1a, b1, s2, t2, w2, b2)


def _edge_agg(tab, dst_arr, src_arr, ew):
    mesh = plsc.VectorSubcoreMesh(core_axis_name="c", subcore_axis_name="s")

    @functools.partial(
        pl.kernel,
        out_type=jax.ShapeDtypeStruct((NRANGE * R, H), jnp.float32),
        mesh=mesh,
        compiler_params=pltpu.CompilerParams(use_tc_tiling_on_sc=False),
        scratch_types=[
            pltpu.VMEM((C,), jnp.int32),
            pltpu.VMEM((C,), jnp.int32),
            pltpu.VMEM((C,), jnp.float32),
            pltpu.VMEM((C,), jnp.int32),
            pltpu.VMEM((C,), jnp.int32),
            pltpu.VMEM((128, H), jnp.float32),
            pltpu.VMEM((160, H), jnp.float32),
            pltpu.VMEM_SHARED((R + 16, H), jnp.float32),
            pltpu.SemaphoreType.DMA,
            pltpu.SemaphoreType.DMA,
        ],
    )
    def k(tab_hbm, dst_hbm, src_hbm, ew_hbm, out_hbm,
          dst_c, src_c, w_c, gidx, dloc, rows, zeros, acc, sem, sem2):
        cid = lax.axis_index("c")
        sid = lax.axis_index("s")
        zero16f = jnp.zeros((16,), jnp.float32)

        def zinit(j, _):
            def zf(f, __):
                zeros[j, pl.ds(f * 16, 16)] = zero16f
                return 0
            return lax.fori_loop(0, H // 16, zf, 0)
        lax.fori_loop(0, 160, zinit, 0)

        nch = 195 + (jnp.right_shift(sid - 5, 31) & 1)

        for rr in range(NRANGE // 2):
            rid = rr * 2 + cid
            lo = rid * R
            for kk in range(TPS // 160):
                pltpu.sync_copy(zeros, acc.at[pl.ds(sid * TPS + kk * 160, 160)])
            plsc.subcore_barrier()

            def chunk_body(ch, _):
                base = (ch * 16 + sid) * C
                pltpu.sync_copy(dst_hbm.at[pl.ds(base, C)], dst_c)
                pltpu.sync_copy(src_hbm.at[pl.ds(base, C)], src_c)
                pltpu.sync_copy(ew_hbm.at[pl.ds(base, C)], w_c)

                def vf(i, __):
                    d = dst_c[pl.ds(i * 16, 16)]
                    s = src_c[pl.ds(i * 16, 16)]
                    dl = d - lo
                    hit = (jnp.right_shift(dl - R, 31)
                           & ~jnp.right_shift(dl, 31) & 1)
                    gidx[pl.ds(i * 16, 16)] = hit * s
                    dloc[pl.ds(i * 16, 16)] = hit * dl + (1 - hit) * DUMMY
                    return 0
                lax.fori_loop(0, C // 16, vf, 0)

                def sub(t, __):
                    tb = t * 128

                    def fire(p, ___):
                        gv = gidx[pl.ds(tb + p * 16, 16)]
                        for l in range(16):
                            pltpu.async_copy(tab_hbm.at[gv[l]],
                                             rows.at[p * 16 + l], sem)
                        return 0
                    lax.fori_loop(0, 8, fire, 0)

                    def dr(p, ___):
                        pltpu.make_async_copy(tab_hbm.at[0],
                                              rows.at[0], sem).wait()
                        return 0
                    lax.fori_loop(0, 128, dr, 0)

                    def group(g, ___):
                        gb = tb + g * 16
                        dl16 = dloc[pl.ds(gb, 16)]
                        w16 = w_c[pl.ds(gb, 16)]
                        for l in range(16):
                            ws = w16[l]
                            r0 = g * 16 + l
                            for f in range(H // 16):
                                rows[r0, pl.ds(f * 16, 16)] = (
                                    rows[r0, pl.ds(f * 16, 16)] * ws)
                        pltpu.async_copy(rows.at[pl.ds(g * 16, 16)],
                                         acc.at[dl16], sem2, add=True)
                        return 0
                    lax.fori_loop(0, 8, group, 0)

                    def dr2(p, ___):
                        pltpu.make_async_copy(rows.at[pl.ds(0, 16)],
                                              acc.at[pl.ds(0, 16)],
                                              sem2).wait()
                        return 0
                    lax.fori_loop(0, 8, dr2, 0)
                    return 0
                lax.fori_loop(0, C // 128, sub, 0)
                return 0
            lax.fori_loop(0, nch, chunk_body, 0)
            plsc.subcore_barrier()

            for kk in range(TPS // 160):
                off = sid * TPS + kk * 160
                pltpu.sync_copy(acc.at[pl.ds(off, 160)],
                                out_hbm.at[pl.ds(lo + off, 160)])
            plsc.subcore_barrier()

    return k(tab, dst_arr, src_arr, ew)


def kernel(node_representations, edges, edge_weights,
           prep_bn1_gamma, prep_bn1_beta, prep_dense1_W, prep_dense1_b,
           prep_bn2_gamma, prep_bn2_beta, prep_dense2_W, prep_dense2_b,
           upd_bn1_gamma, upd_bn1_beta, upd_dense1_W, upd_dense1_b,
           upd_bn2_gamma, upd_bn2_beta, upd_dense2_W, upd_dense2_b):
    f32 = jnp.float32

    prep = _prep_ffn(
        node_representations,
        (prep_bn1_gamma * BN_SCALE)[None, :].astype(f32),
        prep_bn1_beta[None, :],
        prep_dense1_W, prep_dense1_b[None, :],
        (prep_bn2_gamma * BN_SCALE)[None, :].astype(f32),
        prep_bn2_beta[None, :],
        prep_dense2_W, prep_dense2_b[None, :],
    )

    agg = _edge_agg(prep, edges[0], edges[1], edge_weights)

    out = _upd_ffn(
        node_representations, agg,
        (upd_bn1_gamma[:D] * BN_SCALE)[None, :].astype(f32),
        upd_bn1_beta[None, :D],
        (upd_bn1_gamma[D:] * BN_SCALE)[None, :].astype(f32),
        upd_bn1_beta[None, D:],
        upd_dense1_W[:D], upd_dense1_W[D:],
        upd_dense1_b[None, :],
        (upd_bn2_gamma * BN_SCALE)[None, :].astype(f32),
        upd_bn2_beta[None, :],
        upd_dense2_W, upd_dense2_b[None, :],
    )
    return out


# speedup vs baseline: 74.5464x; 74.5197x over previous
import functools
import math

import jax
import jax.numpy as jnp
from jax import lax
from jax.experimental import pallas as pl
from jax.experimental.pallas import tpu as pltpu
from jax.experimental.pallas import tpu_sc as plsc

N = 100000
E = 1600000
D = 128
H = 128
BN_SCALE = 1.0 / math.sqrt(1.0 + 1e-3)

R = 10240
NRANGE = 10
TPS = R // 16
C = 512
DUMMY = R

BLK = 2000


def _prep_body(x_ref, s1_ref, t1_ref, w1_ref, b1_ref, s2_ref, t2_ref,
               w2_ref, b2_ref, o_ref):
    h = x_ref[...] * s1_ref[...] + t1_ref[...]
    h = jax.nn.gelu(jnp.dot(h, w1_ref[...], preferred_element_type=jnp.float32)
                    + b1_ref[...])
    h = h * s2_ref[...] + t2_ref[...]
    o_ref[...] = jax.nn.gelu(
        jnp.dot(h, w2_ref[...], preferred_element_type=jnp.float32) + b2_ref[...])


def _row_spec(rows, cols):
    return pl.BlockSpec((rows, cols), lambda i: (i, 0))


def _full_spec(shape):
    return pl.BlockSpec(shape, lambda i: (0,) * len(shape))


def _prep_ffn(x, s1, t1, w1, b1, s2, t2, w2, b2):
    grid = (N // BLK,)
    return pl.pallas_call(
        _prep_body,
        grid=grid,
        in_specs=[
            _row_spec(BLK, D),
            _full_spec((1, D)), _full_spec((1, D)),
            _full_spec((D, H)), _full_spec((1, H)),
            _full_spec((1, H)), _full_spec((1, H)),
            _full_spec((H, H)), _full_spec((1, H)),
        ],
        out_specs=_row_spec(BLK, H),
        out_shape=jax.ShapeDtypeStruct((N, H), jnp.float32),
    )(x, s1, t1, w1, b1, s2, t2, w2, b2)


def _upd_body(x_ref, a_ref, s1x_ref, t1x_ref, s1a_ref, t1a_ref,
              w1x_ref, w1a_ref, b1_ref, s2_ref, t2_ref, w2_ref, b2_ref, o_ref):
    xs = x_ref[...] * s1x_ref[...] + t1x_ref[...]
    aa = a_ref[...] * s1a_ref[...] + t1a_ref[...]
    h = (jnp.dot(xs, w1x_ref[...], preferred_element_type=jnp.float32)
         + jnp.dot(aa, w1a_ref[...], preferred_element_type=jnp.float32)
         + b1_ref[...])
    h = jax.nn.gelu(h)
    h = h * s2_ref[...] + t2_ref[...]
    h = jax.nn.gelu(jnp.dot(h, w2_ref[...], preferred_element_type=jnp.float32)
                    + b2_ref[...])
    norm = jnp.sqrt(jnp.sum(h * h, axis=-1, keepdims=True))
    o_ref[...] = h / jnp.maximum(norm, 1e-12)


def _upd_ffn(x, agg, s1x, t1x, s1a, t1a, w1x, w1a, b1, s2, t2, w2, b2):
    grid = (N // BLK,)
    return pl.pallas_call(
        _upd_body,
        grid=grid,
        in_specs=[
            _row_spec(BLK, D),
            _row_spec(BLK, H),
            _full_spec((1, D)), _full_spec((1, D)),
            _full_spec((1, H)), _full_spec((1, H)),
            _full_spec((D, H)), _full_spec((H, H)), _full_spec((1, H)),
            _full_spec((1, H)), _full_spec((1, H)),
            _full_spec((H, H)), _full_spec((1, H)),
        ],
        out_specs=_row_spec(BLK, H),
        out_shape=jax.ShapeDtypeStruct((N, H), jnp.float32),
    )(x, agg, s1x, t1x, s1a, t1a, w1x, w1a, b1, s2, t2, w2, b2)


EK = 4000
EG = E // EK


def _tcg_body(idx_ref, w_ref, tab_ref, o_ref):
    def row(j, _):
        i_s = idx_ref[0, 0, j]
        w_s = w_ref[0, 0, j]
        o_ref[pl.ds(j, 1), :] = tab_ref[pl.ds(i_s, 1), :] * w_s
        return 0
    lax.fori_loop(0, EK, row, 0, unroll=8)


def _gather_scale(tab, src3, w3):
    return pl.pallas_call(
        _tcg_body,
        grid=(EG,),
        in_specs=[
            pl.BlockSpec((1, 1, EK), lambda i: (i, 0, 0),
                         memory_space=pltpu.SMEM),
            pl.BlockSpec((1, 1, EK), lambda i: (i, 0, 0),
                         memory_space=pltpu.SMEM),
            pl.BlockSpec((N, H), lambda i: (0, 0)),
        ],
        out_specs=pl.BlockSpec((EK, H), lambda i: (i, 0)),
        out_shape=jax.ShapeDtypeStruct((E, H), jnp.float32),
        compiler_params=pltpu.CompilerParams(
            vmem_limit_bytes=100 * 1024 * 1024),
    )(src3, w3, tab)


def _tcs_body(dst_ref, m_ref, o_ref):
    @pl.when(pl.program_id(0) == 0)
    def _():
        o_ref[...] = jnp.zeros_like(o_ref)

    def row(j, _):
        d = dst_ref[0, 0, j]
        o_ref[pl.ds(d, 1), :] += m_ref[pl.ds(j, 1), :]
        return 0
    lax.fori_loop(0, EK, row, 0, unroll=8)


def _segment_sum(dst3, msgs):
    return pl.pallas_call(
        _tcs_body,
        grid=(EG,),
        in_specs=[
            pl.BlockSpec((1, 1, EK), lambda i: (i, 0, 0),
                         memory_space=pltpu.SMEM),
            pl.BlockSpec((EK, H), lambda i: (i, 0)),
        ],
        out_specs=pl.BlockSpec((N, H), lambda i: (0, 0)),
        out_shape=jax.ShapeDtypeStruct((N, H), jnp.float32),
        compiler_params=pltpu.CompilerParams(
            vmem_limit_bytes=100 * 1024 * 1024),
    )(dst3, msgs)


def _edge_agg(tab, dst_arr, src_arr, ew):
    src3 = src_arr.reshape(EG, 1, EK)
    dst3 = dst_arr.reshape(EG, 1, EK)
    w3 = ew.reshape(EG, 1, EK)
    msgs = _gather_scale(tab, src3, w3)
    return _segment_sum(dst3, msgs)


def kernel(node_representations, edges, edge_weights,
           prep_bn1_gamma, prep_bn1_beta, prep_dense1_W, prep_dense1_b,
           prep_bn2_gamma, prep_bn2_beta, prep_dense2_W, prep_dense2_b,
           upd_bn1_gamma, upd_bn1_beta, upd_dense1_W, upd_dense1_b,
           upd_bn2_gamma, upd_bn2_beta, upd_dense2_W, upd_dense2_b):
    f32 = jnp.float32

    prep = _prep_ffn(
        node_representations,
        (prep_bn1_gamma * BN_SCALE)[None, :].astype(f32),
        prep_bn1_beta[None, :],
        prep_dense1_W, prep_dense1_b[None, :],
        (prep_bn2_gamma * BN_SCALE)[None, :].astype(f32),
        prep_bn2_beta[None, :],
        prep_dense2_W, prep_dense2_b[None, :],
    )

    agg = _edge_agg(prep, edges[0], edges[1], edge_weights)

    out = _upd_ffn(
        node_representations, agg,
        (upd_bn1_gamma[:D] * BN_SCALE)[None, :].astype(f32),
        upd_bn1_beta[None, :D],
        (upd_bn1_gamma[D:] * BN_SCALE)[None, :].astype(f32),
        upd_bn1_beta[None, D:],
        upd_dense1_W[:D], upd_dense1_W[D:],
        upd_dense1_b[None, :],
        (upd_bn2_gamma * BN_SCALE)[None, :].astype(f32),
        upd_bn2_beta[None, :],
        upd_dense2_W, upd_dense2_b[None, :],
    )
    return out


# speedup vs baseline: 74.5519x; 1.0001x over previous
import math

import jax
import jax.numpy as jnp
from jax import lax
from jax.experimental import pallas as pl
from jax.experimental.pallas import tpu as pltpu

N = 100000
E = 1600000
D = 128
H = 128
BN_SCALE = 1.0 / math.sqrt(1.0 + 1e-3)

R = 10240
NRANGE = 10
TPS = R // 16
C = 512
DUMMY = R

BLK = 2000


def _prep_body(x_ref, s1_ref, t1_ref, w1_ref, b1_ref, s2_ref, t2_ref,
               w2_ref, b2_ref, o_ref):
    h = x_ref[...] * s1_ref[...] + t1_ref[...]
    h = jax.nn.gelu(jnp.dot(h, w1_ref[...], preferred_element_type=jnp.float32)
                    + b1_ref[...])
    h = h * s2_ref[...] + t2_ref[...]
    o_ref[...] = jax.nn.gelu(
        jnp.dot(h, w2_ref[...], preferred_element_type=jnp.float32) + b2_ref[...])


def _row_spec(rows, cols):
    return pl.BlockSpec((rows, cols), lambda i: (i, 0))


def _full_spec(shape):
    return pl.BlockSpec(shape, lambda i: (0,) * len(shape))


def _prep_ffn(x, s1, t1, w1, b1, s2, t2, w2, b2):
    grid = (N // BLK,)
    return pl.pallas_call(
        _prep_body,
        grid=grid,
        in_specs=[
            _row_spec(BLK, D),
            _full_spec((1, D)), _full_spec((1, D)),
            _full_spec((D, H)), _full_spec((1, H)),
            _full_spec((1, H)), _full_spec((1, H)),
            _full_spec((H, H)), _full_spec((1, H)),
        ],
        out_specs=_row_spec(BLK, H),
        out_shape=jax.ShapeDtypeStruct((N, H), jnp.float32),
    )(x, s1, t1, w1, b1, s2, t2, w2, b2)


def _upd_body(x_ref, a_ref, s1x_ref, t1x_ref, s1a_ref, t1a_ref,
              w1x_ref, w1a_ref, b1_ref, s2_ref, t2_ref, w2_ref, b2_ref, o_ref):
    xs = x_ref[...] * s1x_ref[...] + t1x_ref[...]
    aa = a_ref[...] * s1a_ref[...] + t1a_ref[...]
    h = (jnp.dot(xs, w1x_ref[...], preferred_element_type=jnp.float32)
         + jnp.dot(aa, w1a_ref[...], preferred_element_type=jnp.float32)
         + b1_ref[...])
    h = jax.nn.gelu(h)
    h = h * s2_ref[...] + t2_ref[...]
    h = jax.nn.gelu(jnp.dot(h, w2_ref[...], preferred_element_type=jnp.float32)
                    + b2_ref[...])
    norm = jnp.sqrt(jnp.sum(h * h, axis=-1, keepdims=True))
    o_ref[...] = h / jnp.maximum(norm, 1e-12)


def _upd_ffn(x, agg, s1x, t1x, s1a, t1a, w1x, w1a, b1, s2, t2, w2, b2):
    grid = (N // BLK,)
    return pl.pallas_call(
        _upd_body,
        grid=grid,
        in_specs=[
            _row_spec(BLK, D),
            _row_spec(BLK, H),
            _full_spec((1, D)), _full_spec((1, D)),
            _full_spec((1, H)), _full_spec((1, H)),
            _full_spec((D, H)), _full_spec((H, H)), _full_spec((1, H)),
            _full_spec((1, H)), _full_spec((1, H)),
            _full_spec((H, H)), _full_spec((1, H)),
        ],
        out_specs=_row_spec(BLK, H),
        out_shape=jax.ShapeDtypeStruct((N, H), jnp.float32),
    )(x, agg, s1x, t1x, s1a, t1a, w1x, w1a, b1, s2, t2, w2, b2)


EK = 4000
EG = E // EK


def _tcg_body(idx_ref, w_ref, tab_ref, o_ref):
    def row(j, _):
        i_s = idx_ref[0, 0, j]
        w_s = w_ref[0, 0, j]
        o_ref[pl.ds(j, 1), :] = tab_ref[pl.ds(i_s, 1), :] * w_s
        return 0
    lax.fori_loop(0, EK, row, 0, unroll=8)


def _gather_scale(tab, src3, w3):
    return pl.pallas_call(
        _tcg_body,
        grid=(EG,),
        in_specs=[
            pl.BlockSpec((1, 1, EK), lambda i: (i, 0, 0),
                         memory_space=pltpu.SMEM),
            pl.BlockSpec((1, 1, EK), lambda i: (i, 0, 0),
                         memory_space=pltpu.SMEM),
            pl.BlockSpec((N, H), lambda i: (0, 0)),
        ],
        out_specs=pl.BlockSpec((EK, H), lambda i: (i, 0)),
        out_shape=jax.ShapeDtypeStruct((E, H), jnp.float32),
        compiler_params=pltpu.CompilerParams(
            vmem_limit_bytes=100 * 1024 * 1024),
    )(src3, w3, tab)


def _tcs_body(dst_ref, m_ref, o_ref):
    @pl.when(pl.program_id(0) == 0)
    def _():
        o_ref[...] = jnp.zeros_like(o_ref)

    def row(j, _):
        d = dst_ref[0, 0, j]
        o_ref[pl.ds(d, 1), :] += m_ref[pl.ds(j, 1), :]
        return 0
    lax.fori_loop(0, EK, row, 0, unroll=8)


def _segment_sum(dst3, msgs):
    return pl.pallas_call(
        _tcs_body,
        grid=(EG,),
        in_specs=[
            pl.BlockSpec((1, 1, EK), lambda i: (i, 0, 0),
                         memory_space=pltpu.SMEM),
            pl.BlockSpec((EK, H), lambda i: (i, 0)),
        ],
        out_specs=pl.BlockSpec((N, H), lambda i: (0, 0)),
        out_shape=jax.ShapeDtypeStruct((N, H), jnp.float32),
        compiler_params=pltpu.CompilerParams(
            vmem_limit_bytes=100 * 1024 * 1024),
    )(dst3, msgs)


def _edge_agg(tab, dst_arr, src_arr, ew):
    src3 = src_arr.reshape(EG, 1, EK)
    dst3 = dst_arr.reshape(EG, 1, EK)
    w3 = ew.reshape(EG, 1, EK)
    msgs = _gather_scale(tab, src3, w3)
    return _segment_sum(dst3, msgs)


def kernel(node_representations, edges, edge_weights,
           prep_bn1_gamma, prep_bn1_beta, prep_dense1_W, prep_dense1_b,
           prep_bn2_gamma, prep_bn2_beta, prep_dense2_W, prep_dense2_b,
           upd_bn1_gamma, upd_bn1_beta, upd_dense1_W, upd_dense1_b,
           upd_bn2_gamma, upd_bn2_beta, upd_dense2_W, upd_dense2_b):
    f32 = jnp.float32

    prep = _prep_ffn(
        node_representations,
        (prep_bn1_gamma * BN_SCALE)[None, :].astype(f32),
        prep_bn1_beta[None, :],
        prep_dense1_W, prep_dense1_b[None, :],
        (prep_bn2_gamma * BN_SCALE)[None, :].astype(f32),
        prep_bn2_beta[None, :],
        prep_dense2_W, prep_dense2_b[None, :],
    )

    agg = _edge_agg(prep, edges[0], edges[1], edge_weights)

    out = _upd_ffn(
        node_representations, agg,
        (upd_bn1_gamma[:D] * BN_SCALE)[None, :].astype(f32),
        upd_bn1_beta[None, :D],
        (upd_bn1_gamma[D:] * BN_SCALE)[None, :].astype(f32),
        upd_bn1_beta[None, D:],
        upd_dense1_W[:D], upd_dense1_W[D:],
        upd_dense1_b[None, :],
        (upd_bn2_gamma * BN_SCALE)[None, :].astype(f32),
        upd_bn2_beta[None, :],
        upd_dense2_W, upd_dense2_b[None, :],
    )
    return out
